# baseline (speedup 1.0000x reference)
import jax
import jax.numpy as jnp
from jax import lax
from jax.experimental import pallas as pl
from jax.experimental.pallas import tpu as pltpu
from jax.experimental.pallas import tpu_sc as plsc

N = 10000
E = 320000
D = 128
NPAD = 10240
NC = 2
NS = 16
L = 16
NW = NC * NS
HALF = D // NC
EP = NW * NPAD
ET32 = EP // NW
ET16 = EP // NS
CG = 128
EBG = 5120
CA = 128
EBA = 2560
R1 = 1024
TRASH = NPAD - 1

_SC_PARAMS = pltpu.CompilerParams(
    needs_layout_passes=False, use_tc_tiling_on_sc=False)


def _sc_mesh():
    return plsc.VectorSubcoreMesh(core_axis_name="c", subcore_axis_name="s")


def _deg_body(src_ref, dst_ref, out_ref, src_v, dst_v, hin, hout, rbuf, slab):
    c = lax.axis_index("c")
    sid = lax.axis_index("s")
    wid = sid * NC + c
    base = wid * ET32
    pltpu.sync_copy(src_ref.at[pl.ds(base, ET32)], src_v)
    pltpu.sync_copy(dst_ref.at[pl.ds(base, ET32)], dst_v)

    zeros16 = jnp.zeros((L,), jnp.float32)

    def zbody(i, _):
        hin[pl.ds(i * L, L)] = zeros16
        hout[pl.ds(i * L, L)] = zeros16
        return 0

    lax.fori_loop(0, NPAD // L, zbody, 0)

    ones16 = jnp.ones((L,), jnp.float32)

    def body(i, _):
        sv = src_v[pl.ds(i * L, L)]
        dv = dst_v[pl.ds(i * L, L)]
        plsc.addupdate_scatter(hout, [sv], ones16)
        plsc.addupdate_scatter(hin, [dv], ones16)
        return 0

    lax.fori_loop(0, ET32 // L, body, 0)
    pltpu.sync_copy(hin, slab.at[sid, pl.ds(0, NPAD)])
    pltpu.sync_copy(hout, slab.at[sid, pl.ds(NPAD, NPAD)])
    plsc.subcore_barrier()
    dchunk = 2 * NPAD // NS
    pltpu.sync_copy(slab.at[:, pl.ds(sid * dchunk, dchunk)], rbuf)

    def red(k, _):
        acc = rbuf[0, pl.ds(k * L, L)]
        for r in range(1, NS):
            acc = acc + rbuf[r, pl.ds(k * L, L)]
        rbuf[0, pl.ds(k * L, L)] = acc
        return 0

    lax.fori_loop(0, dchunk // L, red, 0)
    pltpu.sync_copy(rbuf.at[0], out_ref.at[c, pl.ds(sid * dchunk, dchunk)])


def _deg_partials(src, dst):
    f = pl.kernel(
        _deg_body,
        out_type=jax.ShapeDtypeStruct((NC, 2 * NPAD), jnp.float32),
        mesh=_sc_mesh(),
        scratch_types=[
            pltpu.VMEM((ET32,), jnp.int32),
            pltpu.VMEM((ET32,), jnp.int32),
            pltpu.VMEM((NPAD,), jnp.float32),
            pltpu.VMEM((NPAD,), jnp.float32),
            pltpu.VMEM((NS, 2 * NPAD // NS), jnp.float32),
            pltpu.VMEM_SHARED((NS, 2 * NPAD), jnp.float32),
        ],
        compiler_params=_SC_PARAMS,
    )
    return f(src, dst)


def _s1a_body(x_ref, wa_ref, asrc_ref, adst_ref,
              tgat_ref, s_ref, d_ref):
    x = x_ref[...]
    G = jnp.dot(x, wa_ref[...], preferred_element_type=jnp.float32)
    s = jnp.dot(G, asrc_ref[...], preferred_element_type=jnp.float32)
    d = jnp.dot(G, adst_ref[...], preferred_element_type=jnp.float32)
    P = jnp.exp(s)
    P2 = jnp.exp(0.2 * s)
    Gp = G * P
    Gn = G * P2
    for c in range(NC):
        sl = slice(c * HALF, (c + 1) * HALF)
        tgat_ref[c, 0] = Gp[:, sl]
        tgat_ref[c, 1] = Gn[:, sl]
    s_ref[...] = s
    d_ref[...] = d


def _stage1a(xp, W_gat, a_src, a_dst):
    grid = (NPAD // R1,)
    return pl.pallas_call(
        _s1a_body,
        grid=grid,
        in_specs=[
            pl.BlockSpec((R1, D), lambda i: (i, 0)),
            pl.BlockSpec((D, D), lambda i: (0, 0)),
            pl.BlockSpec((D, 1), lambda i: (0, 0)),
            pl.BlockSpec((D, 1), lambda i: (0, 0)),
        ],
        out_specs=[
            pl.BlockSpec((NC, 2, R1, HALF), lambda i: (0, 0, i, 0)),
            pl.BlockSpec((R1, 1), lambda i: (i, 0)),
            pl.BlockSpec((R1, 1), lambda i: (i, 0)),
        ],
        out_shape=[
            jax.ShapeDtypeStruct((NC, 2, NPAD, HALF), jnp.float32),
            jax.ShapeDtypeStruct((NPAD, 1), jnp.float32),
            jax.ShapeDtypeStruct((NPAD, 1), jnp.float32),
        ],
    )(xp, W_gat, a_src.reshape(D, 1), a_dst.reshape(D, 1))


def _s1b_body(x_ref, wg_ref, degt_ref, tgcn_ref, rin_ref):
    x = x_ref[...]
    H = jnp.dot(x, wg_ref[...], preferred_element_type=jnp.float32)
    dt = degt_ref[...]
    din = jnp.maximum(dt[:, 0:1], 1.0)
    dout = jnp.maximum(dt[:, 1:2], 1.0)
    tgcn_ref[...] = H * lax.rsqrt(dout)
    rin_ref[...] = lax.rsqrt(din)


def _stage1b(xp, W_gcn, degt):
    grid = (NPAD // R1,)
    return pl.pallas_call(
        _s1b_body,
        grid=grid,
        in_specs=[
            pl.BlockSpec((R1, D), lambda i: (i, 0)),
            pl.BlockSpec((D, D), lambda i: (0, 0)),
            pl.BlockSpec((R1, 2), lambda i: (i, 0)),
        ],
        out_specs=[
            pl.BlockSpec((R1, D), lambda i: (i, 0)),
            pl.BlockSpec((R1, 1), lambda i: (i, 0)),
        ],
        out_shape=[
            jax.ShapeDtypeStruct((NPAD, D), jnp.float32),
            jax.ShapeDtypeStruct((NPAD, 1), jnp.float32),
        ],
    )(xp, W_gcn, degt)


def _gcn_body(src_ref, dst_ref, tgcn_ref, agcn_ref,
              srcb, dstb, gb0, gb1, ixg0, ixg1, ixw0, ixw1,
              acc, sem_g0, sem_g1, sem_s0, sem_s1):
    c = lax.axis_index("c")
    sid = lax.axis_index("s")
    wid = sid * NC + c
    base = wid * ET32

    zeros16 = jnp.zeros((L,), jnp.float32)

    def zb_body(r, _):
        for k in range(D // L):
            gb0[r, pl.ds(k * L, L)] = zeros16
        return 0

    lax.fori_loop(0, CG, zb_body, 0)
    rows_per_tile = NPAD // NS
    for k in range(rows_per_tile // CG):
        pltpu.sync_copy(gb0, acc.at[pl.ds(sid * rows_per_tile + k * CG, CG)])
    plsc.subcore_barrier()

    nchb = EBG // CG

    def make_idx(i, ixg, ixw):
        off = i * CG
        for j in range(CG // L):
            o = off + j * L
            ixg[pl.ds(j * L, L)] = srcb[pl.ds(o, L)]
            ixw[0, pl.ds(j * L, L)] = dstb[pl.ds(o, L)]

    def wait_g(gb, ixg, sem):
        pltpu.make_async_copy(tgcn_ref.at[ixg], gb, sem).wait()

    def wait_s(gb, ixw, sem):
        pltpu.make_async_copy(gb, acc.at[ixw.at[0]], sem).wait()

    def block(b, _):
        pltpu.sync_copy(src_ref.at[pl.ds(base + b * EBG, EBG)], srcb)
        pltpu.sync_copy(dst_ref.at[pl.ds(base + b * EBG, EBG)], dstb)

        make_idx(0, ixg0, ixw0)
        pltpu.async_copy(tgcn_ref.at[ixg0], gb0, sem_g0)

        def pair(t, _):
            a = 2 * t

            @pl.when(t > 0)
            def _():
                wait_s(gb1, ixw1, sem_s1)
            make_idx(a + 1, ixg1, ixw1)
            pltpu.async_copy(tgcn_ref.at[ixg1], gb1, sem_g1)
            wait_g(gb0, ixg0, sem_g0)
            pltpu.async_copy(gb0, acc.at[ixw0.at[0]], sem_s0, add=True)
            wait_s(gb0, ixw0, sem_s0)

            @pl.when(a + 2 < nchb)
            def _():
                make_idx(a + 2, ixg0, ixw0)
                pltpu.async_copy(tgcn_ref.at[ixg0], gb0, sem_g0)
            wait_g(gb1, ixg1, sem_g1)
            pltpu.async_copy(gb1, acc.at[ixw1.at[0]], sem_s1, add=True)
            return 0

        lax.fori_loop(0, nchb // 2, pair, 0)
        wait_s(gb1, ixw1, sem_s1)
        return 0

    lax.fori_loop(0, ET32 // EBG, block, 0)
    plsc.subcore_barrier()
    pltpu.sync_copy(acc.at[pl.ds(sid * rows_per_tile, rows_per_tile)],
                    agcn_ref.at[c, pl.ds(sid * rows_per_tile, rows_per_tile)])


def _gcn_pass(src, dst, tgcn):
    f = pl.kernel(
        _gcn_body,
        out_type=jax.ShapeDtypeStruct((NC, NPAD, D), jnp.float32),
        mesh=_sc_mesh(),
        scratch_types=[
            pltpu.VMEM((EBG,), jnp.int32),
            pltpu.VMEM((EBG,), jnp.int32),
            pltpu.VMEM((CG, D), jnp.float32),
            pltpu.VMEM((CG, D), jnp.float32),
            pltpu.VMEM((CG,), jnp.int32),
            pltpu.VMEM((CG,), jnp.int32),
            pltpu.VMEM((1, CG), jnp.int32),
            pltpu.VMEM((1, CG), jnp.int32),
            pltpu.VMEM_SHARED((NPAD, D), jnp.float32),
            pltpu.SemaphoreType.DMA,
            pltpu.SemaphoreType.DMA,
            pltpu.SemaphoreType.DMA,
            pltpu.SemaphoreType.DMA,
        ],
        compiler_params=_SC_PARAMS,
    )
    return f(src, dst, tgcn)


def _gat_body(src_ref, dst_ref, tgat_ref, s_ref, d_ref, agat_ref, sden_ref,
              s_v, d_v, srcb, dstb, gb0, gb1, pv0, pv1, ixg0, ixg1,
              ixw0, ixw1, acc, den_sh,
              sem_g0, sem_g1, sem_s0, sem_s1, sem_d0, sem_d1):
    c = lax.axis_index("c")
    sid = lax.axis_index("s")
    base = sid * ET16

    zeros16 = jnp.zeros((L,), jnp.float32)
    dchunk = 2 * NPAD // NS

    def zs_body(i, _):
        s_v[pl.ds(i * L, L)] = zeros16
        return 0

    lax.fori_loop(0, dchunk // L, zs_body, 0)
    pltpu.sync_copy(s_v.at[pl.ds(0, dchunk)],
                    den_sh.at[pl.ds(sid * dchunk, dchunk)])

    def zb_body(r, _):
        for k in range(HALF // L):
            gb0[r, pl.ds(k * L, L)] = zeros16
        return 0

    lax.fori_loop(0, CA, zb_body, 0)
    rows_per_tile = 2 * NPAD // NS
    for k in range(rows_per_tile // CA):
        pltpu.sync_copy(gb0, acc.at[pl.ds(sid * rows_per_tile + k * CA, CA)])

    pltpu.sync_copy(s_ref, s_v)
    pltpu.sync_copy(d_ref, d_v)
    plsc.subcore_barrier()

    gat_off = c * (2 * NPAD)
    nchb = EBA // CA

    def make_idx(i, ixg, ixw, pv):
        off = i * CA
        for j in range(CA // L):
            o = off + j * L
            sv = srcb[pl.ds(o, L)]
            dv = dstb[pl.ds(o, L)]
            sg = plsc.load_gather(s_v, [sv])
            dg = plsc.load_gather(d_v, [dv])
            negb = (sg + dg) < 0.0
            negi = negb.astype(jnp.int32)
            pv[pl.ds(j * L, L)] = jnp.exp(jnp.where(negb, 0.2 * sg, sg))
            ixg[pl.ds(j * L, L)] = sv + negi * NPAD + gat_off
            ixw[0, pl.ds(j * L, L)] = dv + negi * NPAD

    def wait_g(gb, ixg, sem):
        pltpu.make_async_copy(tgat_ref.at[ixg], gb, sem).wait()

    def wait_sd(gb, pv, ixw, sem_s, sem_d):
        pltpu.make_async_copy(gb, acc.at[ixw.at[0]], sem_s).wait()
        pltpu.make_async_copy(pv, den_sh.at[ixw.at[0]], sem_d).wait()

    def issue_sd(gb, pv, ixw, sem_s, sem_d):
        pltpu.async_copy(gb, acc.at[ixw.at[0]], sem_s, add=True)
        pltpu.async_copy(pv, den_sh.at[ixw.at[0]], sem_d, add=True)

    def block(b, _):
        pltpu.sync_copy(src_ref.at[pl.ds(base + b * EBA, EBA)], srcb)
        pltpu.sync_copy(dst_ref.at[pl.ds(base + b * EBA, EBA)], dstb)

        make_idx(0, ixg0, ixw0, pv0)
        pltpu.async_copy(tgat_ref.at[ixg0], gb0, sem_g0)

        def pair(t, _):
            a = 2 * t

            @pl.when(t > 0)
            def _():
                wait_sd(gb1, pv1, ixw1, sem_s1, sem_d1)
            make_idx(a + 1, ixg1, ixw1, pv1)
            pltpu.async_copy(tgat_ref.at[ixg1], gb1, sem_g1)
            wait_g(gb0, ixg0, sem_g0)
            issue_sd(gb0, pv0, ixw0, sem_s0, sem_d0)
            wait_sd(gb0, pv0, ixw0, sem_s0, sem_d0)

            @pl.when(a + 2 < nchb)
            def _():
                make_idx(a + 2, ixg0, ixw0, pv0)
                pltpu.async_copy(tgat_ref.at[ixg0], gb0, sem_g0)
            wait_g(gb1, ixg1, sem_g1)
            issue_sd(gb1, pv1, ixw1, sem_s1, sem_d1)
            return 0

        lax.fori_loop(0, nchb // 2, pair, 0)
        wait_sd(gb1, pv1, ixw1, sem_s1, sem_d1)
        return 0

    lax.fori_loop(0, ET16 // EBA, block, 0)
    plsc.subcore_barrier()
    pltpu.sync_copy(acc.at[pl.ds(sid * rows_per_tile, rows_per_tile)],
                    agat_ref.at[c, pl.ds(sid * rows_per_tile, rows_per_tile)])
    pltpu.sync_copy(den_sh.at[pl.ds(sid * dchunk, dchunk)],
                    sden_ref.at[c, pl.ds(sid * dchunk, dchunk)])


def _gat_pass(src, dst, tgat_f, svec, dvec):
    f = pl.kernel(
        _gat_body,
        out_type=(
            jax.ShapeDtypeStruct((NC, 2 * NPAD, HALF), jnp.float32),
            jax.ShapeDtypeStruct((NC, 2 * NPAD), jnp.float32),
        ),
        mesh=_sc_mesh(),
        scratch_types=[
            pltpu.VMEM((NPAD,), jnp.float32),
            pltpu.VMEM((NPAD,), jnp.float32),
            pltpu.VMEM((EBA,), jnp.int32),
            pltpu.VMEM((EBA,), jnp.int32),
            pltpu.VMEM((CA, HALF), jnp.float32),
            pltpu.VMEM((CA, HALF), jnp.float32),
            pltpu.VMEM((CA,), jnp.float32),
            pltpu.VMEM((CA,), jnp.float32),
            pltpu.VMEM((CA,), jnp.int32),
            pltpu.VMEM((CA,), jnp.int32),
            pltpu.VMEM((1, CA), jnp.int32),
            pltpu.VMEM((1, CA), jnp.int32),
            pltpu.VMEM_SHARED((2 * NPAD, HALF), jnp.float32),
            pltpu.VMEM_SHARED((2 * NPAD,), jnp.float32),
            pltpu.SemaphoreType.DMA,
            pltpu.SemaphoreType.DMA,
            pltpu.SemaphoreType.DMA,
            pltpu.SemaphoreType.DMA,
            pltpu.SemaphoreType.DMA,
            pltpu.SemaphoreType.DMA,
        ],
        compiler_params=_SC_PARAMS,
    )
    return f(src, dst, tgat_f, svec, dvec)


def _s2a_body(agat_ref, sden_ref, dvec_ref, bgat_ref, y2_ref):
    Apos = jnp.concatenate([agat_ref[0, 0], agat_ref[1, 0]], axis=1)
    Aneg = jnp.concatenate([agat_ref[0, 1], agat_ref[1, 1]], axis=1)
    dv = dvec_ref[...]
    Q = jnp.exp(dv)
    Q2 = jnp.exp(0.2 * dv)
    sden = sden_ref[...]
    denom = jnp.maximum(Q * sden[:, 0:1] + Q2 * sden[:, 1:2], 1e-9)
    y2_ref[...] = (Q * Apos + Q2 * Aneg) / denom + bgat_ref[...]


def _stage2a(agat4, sden_t, dvec, b_gat):
    grid = (NPAD // R1,)
    return pl.pallas_call(
        _s2a_body,
        grid=grid,
        in_specs=[
            pl.BlockSpec((NC, 2, R1, HALF), lambda i: (0, 0, i, 0)),
            pl.BlockSpec((R1, 2), lambda i: (i, 0)),
            pl.BlockSpec((R1, 1), lambda i: (i, 0)),
            pl.BlockSpec((1, D), lambda i: (0, 0)),
        ],
        out_specs=pl.BlockSpec((R1, D), lambda i: (i, 0)),
        out_shape=jax.ShapeDtypeStruct((NPAD, D), jnp.float32),
    )(agat4, sden_t, dvec, b_gat.reshape(1, D))


def _s2b_body(agcn_ref, y2_ref, rin_ref, bgcn_ref, wdec_ref, bdec_ref,
              out_ref):
    y1 = rin_ref[...] * (agcn_ref[0] + agcn_ref[1]) + bgcn_ref[...]
    y2 = y2_ref[...]
    yraw = jnp.concatenate([y1, y2], axis=1)
    y = jnp.dot(yraw, wdec_ref[...], preferred_element_type=jnp.float32) \
        + bdec_ref[...] + y1 + y2
    out_ref[...] = y


def _stage2b(agcn, y2, rin, b_gcn, W_dec, b_dec):
    grid = (NPAD // R1,)
    return pl.pallas_call(
        _s2b_body,
        grid=grid,
        in_specs=[
            pl.BlockSpec((NC, R1, D), lambda i: (0, i, 0)),
            pl.BlockSpec((R1, D), lambda i: (i, 0)),
            pl.BlockSpec((R1, 1), lambda i: (i, 0)),
            pl.BlockSpec((1, D), lambda i: (0, 0)),
            pl.BlockSpec((2 * D, D), lambda i: (0, 0)),
            pl.BlockSpec((1, D), lambda i: (0, 0)),
        ],
        out_specs=pl.BlockSpec((R1, D), lambda i: (i, 0)),
        out_shape=jax.ShapeDtypeStruct((NPAD, D), jnp.float32),
    )(agcn, y2, rin, b_gcn.reshape(1, D), W_dec, b_dec.reshape(1, D))


def kernel(x, edge_index, W_gcn, b_gcn, W_gat, a_src, a_dst, b_gat,
           W_dec, b_dec):
    ei = edge_index.astype(jnp.int32)
    pad_idx = N + jnp.arange(EP - E, dtype=jnp.int32) % (NPAD - N)
    src = jnp.concatenate([ei[0], pad_idx])
    dst = jnp.concatenate([ei[1], pad_idx])
    degp = _deg_partials(src, dst)
    degt = (degp[0] + degp[1]).reshape(2, NPAD).transpose(1, 0)
    xp = jnp.pad(x, ((0, NPAD - N), (0, 0)))
    tgat, svec, dvec = _stage1a(xp, W_gat, a_src, a_dst)
    tgat_f = tgat.reshape(NC * 2 * NPAD, HALF)
    agat, sden = _gat_pass(src, dst, tgat_f,
                           svec.reshape(NPAD), dvec.reshape(NPAD))
    tgcn, rin = _stage1b(xp, W_gcn, degt)
    agcn = _gcn_pass(src, dst, tgcn)
    agat4 = agat.reshape(NC, 2, NPAD, HALF)
    sden_t = sden[0].reshape(2, NPAD).transpose(1, 0)
    y2 = _stage2a(agat4, sden_t, dvec, b_gat)
    y = _stage2b(agcn, y2, rin, b_gcn, W_dec, b_dec)
    return y[:N]


# speedup vs baseline: 1.0351x; 1.0351x over previous
import jax
import jax.numpy as jnp
from jax import lax
from jax.experimental import pallas as pl
from jax.experimental.pallas import tpu as pltpu
from jax.experimental.pallas import tpu_sc as plsc

N = 10000
E = 320000
D = 128
NPAD = 10240
NC = 2
NS = 16
L = 16
NW = NC * NS
HALF = D // NC
EP = NW * NPAD
ET32 = EP // NW
ET16 = EP // NS
CG = 128
EBG = 5120
CA = 128
EBA = 2560
R1 = 1024
TRASH = NPAD - 1

_SC_PARAMS = pltpu.CompilerParams(
    needs_layout_passes=False, use_tc_tiling_on_sc=False)


def _sc_mesh():
    return plsc.VectorSubcoreMesh(core_axis_name="c", subcore_axis_name="s")


def _deg_body(src_ref, dst_ref, out_ref, src_v, dst_v, hin, hout, rbuf, slab):
    c = lax.axis_index("c")
    sid = lax.axis_index("s")
    wid = sid * NC + c
    base = wid * ET32
    pltpu.sync_copy(src_ref.at[pl.ds(base, ET32)], src_v)
    pltpu.sync_copy(dst_ref.at[pl.ds(base, ET32)], dst_v)

    zeros16 = jnp.zeros((L,), jnp.float32)

    def zbody(i, _):
        hin[pl.ds(i * L, L)] = zeros16
        hout[pl.ds(i * L, L)] = zeros16
        return 0

    lax.fori_loop(0, NPAD // L, zbody, 0)

    ones16 = jnp.ones((L,), jnp.float32)

    def body(i, _):
        sv = src_v[pl.ds(i * L, L)]
        dv = dst_v[pl.ds(i * L, L)]
        plsc.addupdate_scatter(hout, [sv], ones16)
        plsc.addupdate_scatter(hin, [dv], ones16)
        return 0

    lax.fori_loop(0, ET32 // L, body, 0)
    pltpu.sync_copy(hin, slab.at[sid, pl.ds(0, NPAD)])
    pltpu.sync_copy(hout, slab.at[sid, pl.ds(NPAD, NPAD)])
    plsc.subcore_barrier()
    dchunk = 2 * NPAD // NS
    pltpu.sync_copy(slab.at[:, pl.ds(sid * dchunk, dchunk)], rbuf)

    def red(k, _):
        acc = rbuf[0, pl.ds(k * L, L)]
        for r in range(1, NS):
            acc = acc + rbuf[r, pl.ds(k * L, L)]
        rbuf[0, pl.ds(k * L, L)] = acc
        return 0

    lax.fori_loop(0, dchunk // L, red, 0)
    pltpu.sync_copy(rbuf.at[0], out_ref.at[c, pl.ds(sid * dchunk, dchunk)])


def _deg_partials(src, dst):
    f = pl.kernel(
        _deg_body,
        out_type=jax.ShapeDtypeStruct((NC, 2 * NPAD), jnp.float32),
        mesh=_sc_mesh(),
        scratch_types=[
            pltpu.VMEM((ET32,), jnp.int32),
            pltpu.VMEM((ET32,), jnp.int32),
            pltpu.VMEM((NPAD,), jnp.float32),
            pltpu.VMEM((NPAD,), jnp.float32),
            pltpu.VMEM((NS, 2 * NPAD // NS), jnp.float32),
            pltpu.VMEM_SHARED((NS, 2 * NPAD), jnp.float32),
        ],
        compiler_params=_SC_PARAMS,
    )
    return f(src, dst)


def _s1_body(x_ref, wg_ref, wa_ref, asrc_ref, adst_ref, degt_ref,
             tgcn_ref, tgat_ref, s_ref, d_ref, rin_ref):
    x = x_ref[...]
    H = jnp.dot(x, wg_ref[...], preferred_element_type=jnp.float32)
    G = jnp.dot(x, wa_ref[...], preferred_element_type=jnp.float32)
    dt = degt_ref[...]
    din = jnp.maximum(dt[:, 0:1], 1.0)
    dout = jnp.maximum(dt[:, 1:2], 1.0)
    rin = lax.rsqrt(din)
    rout = lax.rsqrt(dout)
    s = jnp.dot(G, asrc_ref[...], preferred_element_type=jnp.float32)
    d = jnp.dot(G, adst_ref[...], preferred_element_type=jnp.float32)
    tgcn_ref[...] = H * rout
    P = jnp.exp(s)
    P2 = jnp.exp(0.2 * s)
    Gp = G * P
    Gn = G * P2
    for c in range(NC):
        sl = slice(c * HALF, (c + 1) * HALF)
        tgat_ref[c, 0] = Gp[:, sl]
        tgat_ref[c, 1] = Gn[:, sl]
    s_ref[...] = s
    d_ref[...] = d
    rin_ref[...] = rin


def _stage1(xp, W_gcn, W_gat, a_src, a_dst, degt):
    grid = (NPAD // R1,)
    return pl.pallas_call(
        _s1_body,
        grid=grid,
        in_specs=[
            pl.BlockSpec((R1, D), lambda i: (i, 0)),
            pl.BlockSpec((D, D), lambda i: (0, 0)),
            pl.BlockSpec((D, D), lambda i: (0, 0)),
            pl.BlockSpec((D, 1), lambda i: (0, 0)),
            pl.BlockSpec((D, 1), lambda i: (0, 0)),
            pl.BlockSpec((R1, 2), lambda i: (i, 0)),
        ],
        out_specs=[
            pl.BlockSpec((R1, D), lambda i: (i, 0)),
            pl.BlockSpec((NC, 2, R1, HALF), lambda i: (0, 0, i, 0)),
            pl.BlockSpec((R1, 1), lambda i: (i, 0)),
            pl.BlockSpec((R1, 1), lambda i: (i, 0)),
            pl.BlockSpec((R1, 1), lambda i: (i, 0)),
        ],
        out_shape=[
            jax.ShapeDtypeStruct((NPAD, D), jnp.float32),
            jax.ShapeDtypeStruct((NC, 2, NPAD, HALF), jnp.float32),
            jax.ShapeDtypeStruct((NPAD, 1), jnp.float32),
            jax.ShapeDtypeStruct((NPAD, 1), jnp.float32),
            jax.ShapeDtypeStruct((NPAD, 1), jnp.float32),
        ],
    )(xp, W_gcn, W_gat, a_src.reshape(D, 1), a_dst.reshape(D, 1), degt)


def _gcn_body(src_ref, dst_ref, tgcn_ref, agcn_ref,
              srcb, dstb, gb0, gb1, ixg0, ixg1, ixw0, ixw1,
              acc, sem_g0, sem_g1, sem_s0, sem_s1):
    c = lax.axis_index("c")
    sid = lax.axis_index("s")
    wid = sid * NC + c
    base = wid * ET32

    zeros16 = jnp.zeros((L,), jnp.float32)

    def zb_body(r, _):
        for k in range(D // L):
            gb0[r, pl.ds(k * L, L)] = zeros16
        return 0

    lax.fori_loop(0, CG, zb_body, 0)
    rows_per_tile = NPAD // NS
    for k in range(rows_per_tile // CG):
        pltpu.sync_copy(gb0, acc.at[pl.ds(sid * rows_per_tile + k * CG, CG)])
    plsc.subcore_barrier()

    nchb = EBG // CG

    def make_idx(i, ixg, ixw):
        off = i * CG
        for j in range(CG // L):
            o = off + j * L
            ixg[pl.ds(j * L, L)] = srcb[pl.ds(o, L)]
            ixw[0, pl.ds(j * L, L)] = dstb[pl.ds(o, L)]

    def wait_g(gb, ixg, sem):
        pltpu.make_async_copy(tgcn_ref.at[ixg], gb, sem).wait()

    def wait_s(gb, ixw, sem):
        pltpu.make_async_copy(gb, acc.at[ixw.at[0]], sem).wait()

    def block(b, _):
        pltpu.sync_copy(src_ref.at[pl.ds(base + b * EBG, EBG)], srcb)
        pltpu.sync_copy(dst_ref.at[pl.ds(base + b * EBG, EBG)], dstb)

        make_idx(0, ixg0, ixw0)
        pltpu.async_copy(tgcn_ref.at[ixg0], gb0, sem_g0)

        def pair(t, _):
            a = 2 * t

            @pl.when(t > 0)
            def _():
                wait_s(gb1, ixw1, sem_s1)
            make_idx(a + 1, ixg1, ixw1)
            pltpu.async_copy(tgcn_ref.at[ixg1], gb1, sem_g1)
            wait_g(gb0, ixg0, sem_g0)
            pltpu.async_copy(gb0, acc.at[ixw0.at[0]], sem_s0, add=True)
            wait_s(gb0, ixw0, sem_s0)

            @pl.when(a + 2 < nchb)
            def _():
                make_idx(a + 2, ixg0, ixw0)
                pltpu.async_copy(tgcn_ref.at[ixg0], gb0, sem_g0)
            wait_g(gb1, ixg1, sem_g1)
            pltpu.async_copy(gb1, acc.at[ixw1.at[0]], sem_s1, add=True)
            return 0

        lax.fori_loop(0, nchb // 2, pair, 0)
        wait_s(gb1, ixw1, sem_s1)
        return 0

    lax.fori_loop(0, ET32 // EBG, block, 0)
    plsc.subcore_barrier()
    pltpu.sync_copy(acc.at[pl.ds(sid * rows_per_tile, rows_per_tile)],
                    agcn_ref.at[c, pl.ds(sid * rows_per_tile, rows_per_tile)])


def _gcn_pass(src, dst, tgcn):
    f = pl.kernel(
        _gcn_body,
        out_type=jax.ShapeDtypeStruct((NC, NPAD, D), jnp.float32),
        mesh=_sc_mesh(),
        scratch_types=[
            pltpu.VMEM((EBG,), jnp.int32),
            pltpu.VMEM((EBG,), jnp.int32),
            pltpu.VMEM((CG, D), jnp.float32),
            pltpu.VMEM((CG, D), jnp.float32),
            pltpu.VMEM((CG,), jnp.int32),
            pltpu.VMEM((CG,), jnp.int32),
            pltpu.VMEM((1, CG), jnp.int32),
            pltpu.VMEM((1, CG), jnp.int32),
            pltpu.VMEM_SHARED((NPAD, D), jnp.float32),
            pltpu.SemaphoreType.DMA,
            pltpu.SemaphoreType.DMA,
            pltpu.SemaphoreType.DMA,
            pltpu.SemaphoreType.DMA,
        ],
        compiler_params=_SC_PARAMS,
    )
    return f(src, dst, tgcn)


def _gat_body(src_ref, dst_ref, tgat_ref, s_ref, d_ref, agat_ref, sden_ref,
              s_v, d_v, srcb, dstb, gb0, gb1, pv0, pv1, ixg0, ixg1,
              ixw0, ixw1, acc, den_sh,
              sem_g0, sem_g1, sem_s0, sem_s1, sem_d0, sem_d1):
    c = lax.axis_index("c")
    sid = lax.axis_index("s")
    base = sid * ET16

    zeros16 = jnp.zeros((L,), jnp.float32)
    dchunk = 2 * NPAD // NS

    def zs_body(i, _):
        s_v[pl.ds(i * L, L)] = zeros16
        return 0

    lax.fori_loop(0, dchunk // L, zs_body, 0)
    pltpu.sync_copy(s_v.at[pl.ds(0, dchunk)],
                    den_sh.at[pl.ds(sid * dchunk, dchunk)])

    def zb_body(r, _):
        for k in range(HALF // L):
            gb0[r, pl.ds(k * L, L)] = zeros16
        return 0

    lax.fori_loop(0, CA, zb_body, 0)
    rows_per_tile = 2 * NPAD // NS
    for k in range(rows_per_tile // CA):
        pltpu.sync_copy(gb0, acc.at[pl.ds(sid * rows_per_tile + k * CA, CA)])

    pltpu.sync_copy(s_ref, s_v)
    pltpu.sync_copy(d_ref, d_v)
    plsc.subcore_barrier()

    gat_off = c * (2 * NPAD)
    nchb = EBA // CA

    def make_idx(i, ixg, ixw, pv):
        off = i * CA
        for j in range(CA // L):
            o = off + j * L
            sv = srcb[pl.ds(o, L)]
            dv = dstb[pl.ds(o, L)]
            sg = plsc.load_gather(s_v, [sv])
            dg = plsc.load_gather(d_v, [dv])
            negb = (sg + dg) < 0.0
            negi = negb.astype(jnp.int32)
            pv[pl.ds(j * L, L)] = jnp.exp(jnp.where(negb, 0.2 * sg, sg))
            ixg[pl.ds(j * L, L)] = sv + negi * NPAD + gat_off
            ixw[0, pl.ds(j * L, L)] = dv + negi * NPAD

    def wait_g(gb, ixg, sem):
        pltpu.make_async_copy(tgat_ref.at[ixg], gb, sem).wait()

    def wait_sd(gb, pv, ixw, sem_s, sem_d):
        pltpu.make_async_copy(gb, acc.at[ixw.at[0]], sem_s).wait()
        pltpu.make_async_copy(pv, den_sh.at[ixw.at[0]], sem_d).wait()

    def issue_sd(gb, pv, ixw, sem_s, sem_d):
        pltpu.async_copy(gb, acc.at[ixw.at[0]], sem_s, add=True)
        pltpu.async_copy(pv, den_sh.at[ixw.at[0]], sem_d, add=True)

    def block(b, _):
        pltpu.sync_copy(src_ref.at[pl.ds(base + b * EBA, EBA)], srcb)
        pltpu.sync_copy(dst_ref.at[pl.ds(base + b * EBA, EBA)], dstb)

        make_idx(0, ixg0, ixw0, pv0)
        pltpu.async_copy(tgat_ref.at[ixg0], gb0, sem_g0)

        def pair(t, _):
            a = 2 * t

            @pl.when(t > 0)
            def _():
                wait_sd(gb1, pv1, ixw1, sem_s1, sem_d1)
            make_idx(a + 1, ixg1, ixw1, pv1)
            pltpu.async_copy(tgat_ref.at[ixg1], gb1, sem_g1)
            wait_g(gb0, ixg0, sem_g0)
            issue_sd(gb0, pv0, ixw0, sem_s0, sem_d0)
            wait_sd(gb0, pv0, ixw0, sem_s0, sem_d0)

            @pl.when(a + 2 < nchb)
            def _():
                make_idx(a + 2, ixg0, ixw0, pv0)
                pltpu.async_copy(tgat_ref.at[ixg0], gb0, sem_g0)
            wait_g(gb1, ixg1, sem_g1)
            issue_sd(gb1, pv1, ixw1, sem_s1, sem_d1)
            return 0

        lax.fori_loop(0, nchb // 2, pair, 0)
        wait_sd(gb1, pv1, ixw1, sem_s1, sem_d1)
        return 0

    lax.fori_loop(0, ET16 // EBA, block, 0)
    plsc.subcore_barrier()
    pltpu.sync_copy(acc.at[pl.ds(sid * rows_per_tile, rows_per_tile)],
                    agat_ref.at[c, pl.ds(sid * rows_per_tile, rows_per_tile)])
    pltpu.sync_copy(den_sh.at[pl.ds(sid * dchunk, dchunk)],
                    sden_ref.at[c, pl.ds(sid * dchunk, dchunk)])


def _gat_pass(src, dst, tgat_f, svec, dvec):
    f = pl.kernel(
        _gat_body,
        out_type=(
            jax.ShapeDtypeStruct((NC, 2 * NPAD, HALF), jnp.float32),
            jax.ShapeDtypeStruct((NC, 2 * NPAD), jnp.float32),
        ),
        mesh=_sc_mesh(),
        scratch_types=[
            pltpu.VMEM((NPAD,), jnp.float32),
            pltpu.VMEM((NPAD,), jnp.float32),
            pltpu.VMEM((EBA,), jnp.int32),
            pltpu.VMEM((EBA,), jnp.int32),
            pltpu.VMEM((CA, HALF), jnp.float32),
            pltpu.VMEM((CA, HALF), jnp.float32),
            pltpu.VMEM((CA,), jnp.float32),
            pltpu.VMEM((CA,), jnp.float32),
            pltpu.VMEM((CA,), jnp.int32),
            pltpu.VMEM((CA,), jnp.int32),
            pltpu.VMEM((1, CA), jnp.int32),
            pltpu.VMEM((1, CA), jnp.int32),
            pltpu.VMEM_SHARED((2 * NPAD, HALF), jnp.float32),
            pltpu.VMEM_SHARED((2 * NPAD,), jnp.float32),
            pltpu.SemaphoreType.DMA,
            pltpu.SemaphoreType.DMA,
            pltpu.SemaphoreType.DMA,
            pltpu.SemaphoreType.DMA,
            pltpu.SemaphoreType.DMA,
            pltpu.SemaphoreType.DMA,
        ],
        compiler_params=_SC_PARAMS,
    )
    return f(src, dst, tgat_f, svec, dvec)


def _s2a_body(agat_ref, sden_ref, dvec_ref, bgat_ref, y2_ref):
    Apos = jnp.concatenate([agat_ref[0, 0], agat_ref[1, 0]], axis=1)
    Aneg = jnp.concatenate([agat_ref[0, 1], agat_ref[1, 1]], axis=1)
    dv = dvec_ref[...]
    Q = jnp.exp(dv)
    Q2 = jnp.exp(0.2 * dv)
    sden = sden_ref[...]
    denom = jnp.maximum(Q * sden[:, 0:1] + Q2 * sden[:, 1:2], 1e-9)
    y2_ref[...] = (Q * Apos + Q2 * Aneg) / denom + bgat_ref[...]


def _stage2a(agat4, sden_t, dvec, b_gat):
    grid = (NPAD // R1,)
    return pl.pallas_call(
        _s2a_body,
        grid=grid,
        in_specs=[
            pl.BlockSpec((NC, 2, R1, HALF), lambda i: (0, 0, i, 0)),
            pl.BlockSpec((R1, 2), lambda i: (i, 0)),
            pl.BlockSpec((R1, 1), lambda i: (i, 0)),
            pl.BlockSpec((1, D), lambda i: (0, 0)),
        ],
        out_specs=pl.BlockSpec((R1, D), lambda i: (i, 0)),
        out_shape=jax.ShapeDtypeStruct((NPAD, D), jnp.float32),
    )(agat4, sden_t, dvec, b_gat.reshape(1, D))


def _s2b_body(agcn_ref, y2_ref, rin_ref, bgcn_ref, wdec_ref, bdec_ref,
              out_ref):
    y1 = rin_ref[...] * (agcn_ref[0] + agcn_ref[1]) + bgcn_ref[...]
    y2 = y2_ref[...]
    yraw = jnp.concatenate([y1, y2], axis=1)
    y = jnp.dot(yraw, wdec_ref[...], preferred_element_type=jnp.float32) \
        + bdec_ref[...] + y1 + y2
    out_ref[...] = y


def _stage2b(agcn, y2, rin, b_gcn, W_dec, b_dec):
    grid = (NPAD // R1,)
    return pl.pallas_call(
        _s2b_body,
        grid=grid,
        in_specs=[
            pl.BlockSpec((NC, R1, D), lambda i: (0, i, 0)),
            pl.BlockSpec((R1, D), lambda i: (i, 0)),
            pl.BlockSpec((R1, 1), lambda i: (i, 0)),
            pl.BlockSpec((1, D), lambda i: (0, 0)),
            pl.BlockSpec((2 * D, D), lambda i: (0, 0)),
            pl.BlockSpec((1, D), lambda i: (0, 0)),
        ],
        out_specs=pl.BlockSpec((R1, D), lambda i: (i, 0)),
        out_shape=jax.ShapeDtypeStruct((NPAD, D), jnp.float32),
    )(agcn, y2, rin, b_gcn.reshape(1, D), W_dec, b_dec.reshape(1, D))


def kernel(x, edge_index, W_gcn, b_gcn, W_gat, a_src, a_dst, b_gat,
           W_dec, b_dec):
    ei = edge_index.astype(jnp.int32)
    pad_idx = N + jnp.arange(EP - E, dtype=jnp.int32) % (NPAD - N)
    src = jnp.concatenate([ei[0], pad_idx])
    dst = jnp.concatenate([ei[1], pad_idx])
    degp = _deg_partials(src, dst)
    degt = (degp[0] + degp[1]).reshape(2, NPAD).transpose(1, 0)
    xp = jnp.pad(x, ((0, NPAD - N), (0, 0)))
    tgcn, tgat, svec, dvec, rin = _stage1(xp, W_gcn, W_gat, a_src, a_dst, degt)
    tgat_f = tgat.reshape(NC * 2 * NPAD, HALF)
    agat, sden = _gat_pass(src, dst, tgat_f,
                           svec.reshape(NPAD), dvec.reshape(NPAD))
    agcn = _gcn_pass(src, dst, tgcn)
    agat4 = agat.reshape(NC, 2, NPAD, HALF)
    sden_t = sden[0].reshape(2, NPAD).transpose(1, 0)
    y2 = _stage2a(agat4, sden_t, dvec, b_gat)
    y = _stage2b(agcn, y2, rin, b_gcn, W_dec, b_dec)
    return y[:N]


# speedup vs baseline: 1.0646x; 1.0285x over previous
import jax
import jax.numpy as jnp
from jax import lax
from jax.experimental import pallas as pl
from jax.experimental.pallas import tpu as pltpu
from jax.experimental.pallas import tpu_sc as plsc

N = 10000
E = 320000
D = 128
NPAD = 10240
NC = 2
NS = 16
L = 16
NW = NC * NS
HALF = D // NC
EP = NW * NPAD
ET32 = EP // NW
ET16 = EP // NS
CG = 128
EBG = 5120
CA = 128
EBA = 5120
R1 = 1024
TRASH = NPAD - 1

_SC_PARAMS = pltpu.CompilerParams(
    needs_layout_passes=False, use_tc_tiling_on_sc=False)


def _sc_mesh():
    return plsc.VectorSubcoreMesh(core_axis_name="c", subcore_axis_name="s")


def _deg_body(src_ref, dst_ref, out_ref, src_v, dst_v, hin, hout, rbuf, slab):
    c = lax.axis_index("c")
    sid = lax.axis_index("s")
    wid = sid * NC + c
    base = wid * ET32
    pltpu.sync_copy(src_ref.at[pl.ds(base, ET32)], src_v)
    pltpu.sync_copy(dst_ref.at[pl.ds(base, ET32)], dst_v)

    zeros16 = jnp.zeros((L,), jnp.float32)

    def zbody(i, _):
        hin[pl.ds(i * L, L)] = zeros16
        hout[pl.ds(i * L, L)] = zeros16
        return 0

    lax.fori_loop(0, NPAD // L, zbody, 0)

    ones16 = jnp.ones((L,), jnp.float32)

    def body(i, _):
        sv = src_v[pl.ds(i * L, L)]
        dv = dst_v[pl.ds(i * L, L)]
        plsc.addupdate_scatter(hout, [sv], ones16)
        plsc.addupdate_scatter(hin, [dv], ones16)
        return 0

    lax.fori_loop(0, ET32 // L, body, 0)
    pltpu.sync_copy(hin, slab.at[sid, pl.ds(0, NPAD)])
    pltpu.sync_copy(hout, slab.at[sid, pl.ds(NPAD, NPAD)])
    plsc.subcore_barrier()
    dchunk = 2 * NPAD // NS
    pltpu.sync_copy(slab.at[:, pl.ds(sid * dchunk, dchunk)], rbuf)

    def red(k, _):
        acc = rbuf[0, pl.ds(k * L, L)]
        for r in range(1, NS):
            acc = acc + rbuf[r, pl.ds(k * L, L)]
        rbuf[0, pl.ds(k * L, L)] = acc
        return 0

    lax.fori_loop(0, dchunk // L, red, 0)
    pltpu.sync_copy(rbuf.at[0], out_ref.at[c, pl.ds(sid * dchunk, dchunk)])


def _deg_partials(src, dst):
    f = pl.kernel(
        _deg_body,
        out_type=jax.ShapeDtypeStruct((NC, 2 * NPAD), jnp.float32),
        mesh=_sc_mesh(),
        scratch_types=[
            pltpu.VMEM((ET32,), jnp.int32),
            pltpu.VMEM((ET32,), jnp.int32),
            pltpu.VMEM((NPAD,), jnp.float32),
            pltpu.VMEM((NPAD,), jnp.float32),
            pltpu.VMEM((NS, 2 * NPAD // NS), jnp.float32),
            pltpu.VMEM_SHARED((NS, 2 * NPAD), jnp.float32),
        ],
        compiler_params=_SC_PARAMS,
    )
    return f(src, dst)


def _s1_body(x_ref, wg_ref, wa_ref, asrc_ref, adst_ref, degt_ref,
             tgcn_ref, tgat_ref, s_ref, d_ref, rin_ref):
    x = x_ref[...]
    H = jnp.dot(x, wg_ref[...], preferred_element_type=jnp.float32)
    G = jnp.dot(x, wa_ref[...], preferred_element_type=jnp.float32)
    dt = degt_ref[...]
    din = jnp.maximum(dt[:, 0:1], 1.0)
    dout = jnp.maximum(dt[:, 1:2], 1.0)
    rin = lax.rsqrt(din)
    rout = lax.rsqrt(dout)
    s = jnp.dot(G, asrc_ref[...], preferred_element_type=jnp.float32)
    d = jnp.dot(G, adst_ref[...], preferred_element_type=jnp.float32)
    tgcn_ref[...] = H * rout
    P = jnp.exp(s)
    P2 = jnp.exp(0.2 * s)
    Gp = G * P
    Gn = G * P2
    for c in range(NC):
        sl = slice(c * HALF, (c + 1) * HALF)
        tgat_ref[c, 0] = Gp[:, sl]
        tgat_ref[c, 1] = Gn[:, sl]
    s_ref[...] = s
    d_ref[...] = d
    rin_ref[...] = rin


def _stage1(xp, W_gcn, W_gat, a_src, a_dst, degt):
    grid = (NPAD // R1,)
    return pl.pallas_call(
        _s1_body,
        grid=grid,
        in_specs=[
            pl.BlockSpec((R1, D), lambda i: (i, 0)),
            pl.BlockSpec((D, D), lambda i: (0, 0)),
            pl.BlockSpec((D, D), lambda i: (0, 0)),
            pl.BlockSpec((D, 1), lambda i: (0, 0)),
            pl.BlockSpec((D, 1), lambda i: (0, 0)),
            pl.BlockSpec((R1, 2), lambda i: (i, 0)),
        ],
        out_specs=[
            pl.BlockSpec((R1, D), lambda i: (i, 0)),
            pl.BlockSpec((NC, 2, R1, HALF), lambda i: (0, 0, i, 0)),
            pl.BlockSpec((R1, 1), lambda i: (i, 0)),
            pl.BlockSpec((R1, 1), lambda i: (i, 0)),
            pl.BlockSpec((R1, 1), lambda i: (i, 0)),
        ],
        out_shape=[
            jax.ShapeDtypeStruct((NPAD, D), jnp.float32),
            jax.ShapeDtypeStruct((NC, 2, NPAD, HALF), jnp.float32),
            jax.ShapeDtypeStruct((NPAD, 1), jnp.float32),
            jax.ShapeDtypeStruct((NPAD, 1), jnp.float32),
            jax.ShapeDtypeStruct((NPAD, 1), jnp.float32),
        ],
    )(xp, W_gcn, W_gat, a_src.reshape(D, 1), a_dst.reshape(D, 1), degt)


def _gcn_body(src_ref, dst_ref, tgcn_ref, agcn_ref,
              srcb, dstb, gb0, gb1, ixg0, ixg1, ixw0, ixw1,
              acc, sem_g0, sem_g1, sem_s0, sem_s1):
    c = lax.axis_index("c")
    sid = lax.axis_index("s")
    wid = sid * NC + c
    base = wid * ET32

    zeros16 = jnp.zeros((L,), jnp.float32)

    def zb_body(r, _):
        for k in range(D // L):
            gb0[r, pl.ds(k * L, L)] = zeros16
        return 0

    lax.fori_loop(0, CG, zb_body, 0)
    rows_per_tile = NPAD // NS
    for k in range(rows_per_tile // CG):
        pltpu.sync_copy(gb0, acc.at[pl.ds(sid * rows_per_tile + k * CG, CG)])
    plsc.subcore_barrier()

    nchb = EBG // CG

    def make_idx(i, ixg, ixw):
        off = i * CG
        for j in range(CG // L):
            o = off + j * L
            ixg[pl.ds(j * L, L)] = srcb[pl.ds(o, L)]
            ixw[0, pl.ds(j * L, L)] = dstb[pl.ds(o, L)]

    def wait_g(gb, ixg, sem):
        pltpu.make_async_copy(tgcn_ref.at[ixg], gb, sem).wait()

    def wait_s(gb, ixw, sem):
        pltpu.make_async_copy(gb, acc.at[ixw.at[0]], sem).wait()

    def block(b, _):
        pltpu.sync_copy(src_ref.at[pl.ds(base + b * EBG, EBG)], srcb)
        pltpu.sync_copy(dst_ref.at[pl.ds(base + b * EBG, EBG)], dstb)

        make_idx(0, ixg0, ixw0)
        pltpu.async_copy(tgcn_ref.at[ixg0], gb0, sem_g0)

        def pair(t, _):
            a = 2 * t

            @pl.when(t > 0)
            def _():
                wait_s(gb1, ixw1, sem_s1)
            make_idx(a + 1, ixg1, ixw1)
            pltpu.async_copy(tgcn_ref.at[ixg1], gb1, sem_g1)
            wait_g(gb0, ixg0, sem_g0)
            pltpu.async_copy(gb0, acc.at[ixw0.at[0]], sem_s0, add=True)
            wait_s(gb0, ixw0, sem_s0)

            @pl.when(a + 2 < nchb)
            def _():
                make_idx(a + 2, ixg0, ixw0)
                pltpu.async_copy(tgcn_ref.at[ixg0], gb0, sem_g0)
            wait_g(gb1, ixg1, sem_g1)
            pltpu.async_copy(gb1, acc.at[ixw1.at[0]], sem_s1, add=True)
            return 0

        lax.fori_loop(0, nchb // 2, pair, 0)
        wait_s(gb1, ixw1, sem_s1)
        return 0

    lax.fori_loop(0, ET32 // EBG, block, 0)
    plsc.subcore_barrier()
    pltpu.sync_copy(acc.at[pl.ds(sid * rows_per_tile, rows_per_tile)],
                    agcn_ref.at[c, pl.ds(sid * rows_per_tile, rows_per_tile)])


def _gcn_pass(src, dst, tgcn):
    f = pl.kernel(
        _gcn_body,
        out_type=jax.ShapeDtypeStruct((NC, NPAD, D), jnp.float32),
        mesh=_sc_mesh(),
        scratch_types=[
            pltpu.VMEM((EBG,), jnp.int32),
            pltpu.VMEM((EBG,), jnp.int32),
            pltpu.VMEM((CG, D), jnp.float32),
            pltpu.VMEM((CG, D), jnp.float32),
            pltpu.VMEM((CG,), jnp.int32),
            pltpu.VMEM((CG,), jnp.int32),
            pltpu.VMEM((1, CG), jnp.int32),
            pltpu.VMEM((1, CG), jnp.int32),
            pltpu.VMEM_SHARED((NPAD, D), jnp.float32),
            pltpu.SemaphoreType.DMA,
            pltpu.SemaphoreType.DMA,
            pltpu.SemaphoreType.DMA,
            pltpu.SemaphoreType.DMA,
        ],
        compiler_params=_SC_PARAMS,
    )
    return f(src, dst, tgcn)


def _gat_body(src_ref, dst_ref, tgat_ref, s_ref, d_ref, agat_ref, sden_ref,
              s_v, d_v, srcb, dstb, gb0, gb1, pv0, pv1, ixg0, ixg1,
              ixw0, ixw1, acc, den_sh,
              sem_g0, sem_g1, sem_s0, sem_s1, sem_d0, sem_d1):
    c = lax.axis_index("c")
    sid = lax.axis_index("s")
    base = sid * ET16

    zeros16 = jnp.zeros((L,), jnp.float32)
    dchunk = 2 * NPAD // NS

    def zs_body(i, _):
        s_v[pl.ds(i * L, L)] = zeros16
        return 0

    lax.fori_loop(0, dchunk // L, zs_body, 0)
    pltpu.sync_copy(s_v.at[pl.ds(0, dchunk)],
                    den_sh.at[pl.ds(sid * dchunk, dchunk)])

    def zb_body(r, _):
        for k in range(HALF // L):
            gb0[r, pl.ds(k * L, L)] = zeros16
        return 0

    lax.fori_loop(0, CA, zb_body, 0)
    rows_per_tile = 2 * NPAD // NS
    for k in range(rows_per_tile // CA):
        pltpu.sync_copy(gb0, acc.at[pl.ds(sid * rows_per_tile + k * CA, CA)])

    pltpu.sync_copy(s_ref, s_v)
    pltpu.sync_copy(d_ref, d_v)
    plsc.subcore_barrier()

    gat_off = c * (2 * NPAD)
    nchb = EBA // CA

    def make_idx(i, ixg, ixw, pv):
        off = i * CA
        for j in range(CA // L):
            o = off + j * L
            sv = srcb[pl.ds(o, L)]
            dv = dstb[pl.ds(o, L)]
            sg = plsc.load_gather(s_v, [sv])
            dg = plsc.load_gather(d_v, [dv])
            negb = (sg + dg) < 0.0
            negi = negb.astype(jnp.int32)
            pv[pl.ds(j * L, L)] = jnp.exp(jnp.where(negb, 0.2 * sg, sg))
            ixg[pl.ds(j * L, L)] = sv + negi * NPAD + gat_off
            ixw[0, pl.ds(j * L, L)] = dv + negi * NPAD

    def wait_g(gb, ixg, sem):
        pltpu.make_async_copy(tgat_ref.at[ixg], gb, sem).wait()

    def wait_sd(gb, pv, ixw, sem_s, sem_d):
        pltpu.make_async_copy(gb, acc.at[ixw.at[0]], sem_s).wait()
        pltpu.make_async_copy(pv, den_sh.at[ixw.at[0]], sem_d).wait()

    def issue_sd(gb, pv, ixw, sem_s, sem_d):
        pltpu.async_copy(gb, acc.at[ixw.at[0]], sem_s, add=True)
        pltpu.async_copy(pv, den_sh.at[ixw.at[0]], sem_d, add=True)

    def block(b, _):
        pltpu.sync_copy(src_ref.at[pl.ds(base + b * EBA, EBA)], srcb)
        pltpu.sync_copy(dst_ref.at[pl.ds(base + b * EBA, EBA)], dstb)

        make_idx(0, ixg0, ixw0, pv0)
        pltpu.async_copy(tgat_ref.at[ixg0], gb0, sem_g0)

        def pair(t, _):
            a = 2 * t

            @pl.when(t > 0)
            def _():
                wait_sd(gb1, pv1, ixw1, sem_s1, sem_d1)
            make_idx(a + 1, ixg1, ixw1, pv1)
            pltpu.async_copy(tgat_ref.at[ixg1], gb1, sem_g1)
            wait_g(gb0, ixg0, sem_g0)
            issue_sd(gb0, pv0, ixw0, sem_s0, sem_d0)
            wait_sd(gb0, pv0, ixw0, sem_s0, sem_d0)

            @pl.when(a + 2 < nchb)
            def _():
                make_idx(a + 2, ixg0, ixw0, pv0)
                pltpu.async_copy(tgat_ref.at[ixg0], gb0, sem_g0)
            wait_g(gb1, ixg1, sem_g1)
            issue_sd(gb1, pv1, ixw1, sem_s1, sem_d1)
            return 0

        lax.fori_loop(0, nchb // 2, pair, 0)
        wait_sd(gb1, pv1, ixw1, sem_s1, sem_d1)
        return 0

    lax.fori_loop(0, ET16 // EBA, block, 0)
    plsc.subcore_barrier()
    pltpu.sync_copy(acc.at[pl.ds(sid * rows_per_tile, rows_per_tile)],
                    agat_ref.at[c, pl.ds(sid * rows_per_tile, rows_per_tile)])
    pltpu.sync_copy(den_sh.at[pl.ds(sid * dchunk, dchunk)],
                    sden_ref.at[c, pl.ds(sid * dchunk, dchunk)])


def _gat_pass(src, dst, tgat_f, svec, dvec):
    f = pl.kernel(
        _gat_body,
        out_type=(
            jax.ShapeDtypeStruct((NC, 2 * NPAD, HALF), jnp.float32),
            jax.ShapeDtypeStruct((NC, 2 * NPAD), jnp.float32),
        ),
        mesh=_sc_mesh(),
        scratch_types=[
            pltpu.VMEM((NPAD,), jnp.float32),
            pltpu.VMEM((NPAD,), jnp.float32),
            pltpu.VMEM((EBA,), jnp.int32),
            pltpu.VMEM((EBA,), jnp.int32),
            pltpu.VMEM((CA, HALF), jnp.float32),
            pltpu.VMEM((CA, HALF), jnp.float32),
            pltpu.VMEM((CA,), jnp.float32),
            pltpu.VMEM((CA,), jnp.float32),
            pltpu.VMEM((CA,), jnp.int32),
            pltpu.VMEM((CA,), jnp.int32),
            pltpu.VMEM((1, CA), jnp.int32),
            pltpu.VMEM((1, CA), jnp.int32),
            pltpu.VMEM_SHARED((2 * NPAD, HALF), jnp.float32),
            pltpu.VMEM_SHARED((2 * NPAD,), jnp.float32),
            pltpu.SemaphoreType.DMA,
            pltpu.SemaphoreType.DMA,
            pltpu.SemaphoreType.DMA,
            pltpu.SemaphoreType.DMA,
            pltpu.SemaphoreType.DMA,
            pltpu.SemaphoreType.DMA,
        ],
        compiler_params=_SC_PARAMS,
    )
    return f(src, dst, tgat_f, svec, dvec)


def _s2a_body(agat_ref, sden_ref, dvec_ref, bgat_ref, y2_ref):
    Apos = jnp.concatenate([agat_ref[0, 0], agat_ref[1, 0]], axis=1)
    Aneg = jnp.concatenate([agat_ref[0, 1], agat_ref[1, 1]], axis=1)
    dv = dvec_ref[...]
    Q = jnp.exp(dv)
    Q2 = jnp.exp(0.2 * dv)
    sden = sden_ref[...]
    denom = jnp.maximum(Q * sden[:, 0:1] + Q2 * sden[:, 1:2], 1e-9)
    y2_ref[...] = (Q * Apos + Q2 * Aneg) / denom + bgat_ref[...]


def _stage2a(agat4, sden_t, dvec, b_gat):
    grid = (NPAD // R1,)
    return pl.pallas_call(
        _s2a_body,
        grid=grid,
        in_specs=[
            pl.BlockSpec((NC, 2, R1, HALF), lambda i: (0, 0, i, 0)),
            pl.BlockSpec((R1, 2), lambda i: (i, 0)),
            pl.BlockSpec((R1, 1), lambda i: (i, 0)),
            pl.BlockSpec((1, D), lambda i: (0, 0)),
        ],
        out_specs=pl.BlockSpec((R1, D), lambda i: (i, 0)),
        out_shape=jax.ShapeDtypeStruct((NPAD, D), jnp.float32),
    )(agat4, sden_t, dvec, b_gat.reshape(1, D))


def _s2b_body(agcn_ref, y2_ref, rin_ref, bgcn_ref, wdec_ref, bdec_ref,
              out_ref):
    y1 = rin_ref[...] * (agcn_ref[0] + agcn_ref[1]) + bgcn_ref[...]
    y2 = y2_ref[...]
    yraw = jnp.concatenate([y1, y2], axis=1)
    y = jnp.dot(yraw, wdec_ref[...], preferred_element_type=jnp.float32) \
        + bdec_ref[...] + y1 + y2
    out_ref[...] = y


def _stage2b(agcn, y2, rin, b_gcn, W_dec, b_dec):
    grid = (NPAD // R1,)
    return pl.pallas_call(
        _s2b_body,
        grid=grid,
        in_specs=[
            pl.BlockSpec((NC, R1, D), lambda i: (0, i, 0)),
            pl.BlockSpec((R1, D), lambda i: (i, 0)),
            pl.BlockSpec((R1, 1), lambda i: (i, 0)),
            pl.BlockSpec((1, D), lambda i: (0, 0)),
            pl.BlockSpec((2 * D, D), lambda i: (0, 0)),
            pl.BlockSpec((1, D), lambda i: (0, 0)),
        ],
        out_specs=pl.BlockSpec((R1, D), lambda i: (i, 0)),
        out_shape=jax.ShapeDtypeStruct((N, D), jnp.float32),
    )(agcn, y2, rin, b_gcn.reshape(1, D), W_dec, b_dec.reshape(1, D))


def kernel(x, edge_index, W_gcn, b_gcn, W_gat, a_src, a_dst, b_gat,
           W_dec, b_dec):
    ei = edge_index.astype(jnp.int32)
    pad_idx = N + jnp.arange(EP - E, dtype=jnp.int32) % (NPAD - N)
    src = jnp.concatenate([ei[0], pad_idx])
    dst = jnp.concatenate([ei[1], pad_idx])
    degp = _deg_partials(src, dst)
    degt = (degp[0] + degp[1]).reshape(2, NPAD).transpose(1, 0)
    xp = jnp.pad(x, ((0, NPAD - N), (0, 0)))
    tgcn, tgat, svec, dvec, rin = _stage1(xp, W_gcn, W_gat, a_src, a_dst, degt)
    tgat_f = tgat.reshape(NC * 2 * NPAD, HALF)
    agat, sden = _gat_pass(src, dst, tgat_f,
                           svec.reshape(NPAD), dvec.reshape(NPAD))
    agcn = _gcn_pass(src, dst, tgcn)
    agat4 = agat.reshape(NC, 2, NPAD, HALF)
    sden_t = sden[0].reshape(2, NPAD).transpose(1, 0)
    y2 = _stage2a(agat4, sden_t, dvec, b_gat)
    return _stage2b(agcn, y2, rin, b_gcn, W_dec, b_dec)


# speedup vs baseline: 1.0662x; 1.0015x over previous
import jax
import jax.numpy as jnp
from jax import lax
from jax.experimental import pallas as pl
from jax.experimental.pallas import tpu as pltpu
from jax.experimental.pallas import tpu_sc as plsc

N = 10000
E = 320000
D = 128
NPAD = 10240
NC = 2
NS = 16
L = 16
NW = NC * NS
HALF = D // NC
EP = NW * NPAD
ET32 = EP // NW
ET16 = EP // NS
CG = 128
EBG = 5120
CA = 128
EBA = 5120
R1 = 1024

_SC_PARAMS = pltpu.CompilerParams(
    needs_layout_passes=False, use_tc_tiling_on_sc=False)


def _sc_mesh():
    return plsc.VectorSubcoreMesh(core_axis_name="c", subcore_axis_name="s")


def _deg_body(src_ref, dst_ref, out_ref, src_v, dst_v, hin, hout, rbuf, slab):
    c = lax.axis_index("c")
    sid = lax.axis_index("s")
    wid = sid * NC + c
    base = wid * ET32
    pltpu.sync_copy(src_ref.at[pl.ds(base, ET32)], src_v)
    pltpu.sync_copy(dst_ref.at[pl.ds(base, ET32)], dst_v)

    zeros16 = jnp.zeros((L,), jnp.float32)

    def zbody(i, _):
        hin[pl.ds(i * L, L)] = zeros16
        hout[pl.ds(i * L, L)] = zeros16
        return 0

    lax.fori_loop(0, NPAD // L, zbody, 0)

    ones16 = jnp.ones((L,), jnp.float32)

    def body(i, _):
        sv = src_v[pl.ds(i * L, L)]
        dv = dst_v[pl.ds(i * L, L)]
        plsc.addupdate_scatter(hout, [sv], ones16)
        plsc.addupdate_scatter(hin, [dv], ones16)
        return 0

    lax.fori_loop(0, ET32 // L, body, 0)
    pltpu.sync_copy(hin, slab.at[sid, pl.ds(0, NPAD)])
    pltpu.sync_copy(hout, slab.at[sid, pl.ds(NPAD, NPAD)])
    plsc.subcore_barrier()
    dchunk = 2 * NPAD // NS
    pltpu.sync_copy(slab.at[:, pl.ds(sid * dchunk, dchunk)], rbuf)

    def red(k, _):
        acc = rbuf[0, pl.ds(k * L, L)]
        for r in range(1, NS):
            acc = acc + rbuf[r, pl.ds(k * L, L)]
        rbuf[0, pl.ds(k * L, L)] = acc
        return 0

    lax.fori_loop(0, dchunk // L, red, 0)
    pltpu.sync_copy(rbuf.at[0], out_ref.at[c, pl.ds(sid * dchunk, dchunk)])


def _deg_partials(src, dst):
    f = pl.kernel(
        _deg_body,
        out_type=jax.ShapeDtypeStruct((NC, 2 * NPAD), jnp.float32),
        mesh=_sc_mesh(),
        scratch_types=[
            pltpu.VMEM((ET32,), jnp.int32),
            pltpu.VMEM((ET32,), jnp.int32),
            pltpu.VMEM((NPAD,), jnp.float32),
            pltpu.VMEM((NPAD,), jnp.float32),
            pltpu.VMEM((NS, 2 * NPAD // NS), jnp.float32),
            pltpu.VMEM_SHARED((NS, 2 * NPAD), jnp.float32),
        ],
        compiler_params=_SC_PARAMS,
    )
    return f(src, dst)


def _s1_body(x_ref, wg_ref, wa_ref, asrc_ref, adst_ref, degt_ref,
             tgcn_ref, tgat_ref, s_ref, d_ref, rin_ref):
    x = x_ref[...]
    H = jnp.dot(x, wg_ref[...], preferred_element_type=jnp.float32)
    G = jnp.dot(x, wa_ref[...], preferred_element_type=jnp.float32)
    dt = degt_ref[...]
    din = jnp.maximum(dt[:, 0:1], 1.0)
    dout = jnp.maximum(dt[:, 1:2], 1.0)
    rin = lax.rsqrt(din)
    rout = lax.rsqrt(dout)
    s = jnp.dot(G, asrc_ref[...], preferred_element_type=jnp.float32)
    d = jnp.dot(G, adst_ref[...], preferred_element_type=jnp.float32)
    tgcn_ref[...] = H * rout
    P = jnp.exp(s)
    P2 = jnp.exp(0.2 * s)
    Gp = G * P
    Gn = G * P2
    for c in range(NC):
        sl = slice(c * HALF, (c + 1) * HALF)
        tgat_ref[c, 0] = Gp[:, sl]
        tgat_ref[c, 1] = Gn[:, sl]
    s_ref[...] = s
    d_ref[...] = d
    rin_ref[...] = rin


def _stage1(xp, W_gcn, W_gat, a_src, a_dst, degt):
    grid = (NPAD // R1,)
    return pl.pallas_call(
        _s1_body,
        grid=grid,
        in_specs=[
            pl.BlockSpec((R1, D), lambda i: (i, 0)),
            pl.BlockSpec((D, D), lambda i: (0, 0)),
            pl.BlockSpec((D, D), lambda i: (0, 0)),
            pl.BlockSpec((D, 1), lambda i: (0, 0)),
            pl.BlockSpec((D, 1), lambda i: (0, 0)),
            pl.BlockSpec((R1, 2), lambda i: (i, 0)),
        ],
        out_specs=[
            pl.BlockSpec((R1, D), lambda i: (i, 0)),
            pl.BlockSpec((NC, 2, R1, HALF), lambda i: (0, 0, i, 0)),
            pl.BlockSpec((R1, 1), lambda i: (i, 0)),
            pl.BlockSpec((R1, 1), lambda i: (i, 0)),
            pl.BlockSpec((R1, 1), lambda i: (i, 0)),
        ],
        out_shape=[
            jax.ShapeDtypeStruct((NPAD, D), jnp.float32),
            jax.ShapeDtypeStruct((NC, 2, NPAD, HALF), jnp.float32),
            jax.ShapeDtypeStruct((NPAD, 1), jnp.float32),
            jax.ShapeDtypeStruct((NPAD, 1), jnp.float32),
            jax.ShapeDtypeStruct((NPAD, 1), jnp.float32),
        ],
    )(xp, W_gcn, W_gat, a_src.reshape(D, 1), a_dst.reshape(D, 1), degt)


def _gcn_body(src_ref, dst_ref, tgcn_ref, agcn_ref,
              srcb, dstb, gb0, gb1, ixg0, ixg1, ixw0, ixw1,
              acc, sem_g0, sem_g1, sem_s0, sem_s1):
    c = lax.axis_index("c")
    sid = lax.axis_index("s")
    wid = sid * NC + c
    base = wid * ET32

    zeros16 = jnp.zeros((L,), jnp.float32)

    def zb_body(r, _):
        for k in range(D // L):
            gb0[r, pl.ds(k * L, L)] = zeros16
        return 0

    lax.fori_loop(0, CG, zb_body, 0)
    rows_per_tile = NPAD // NS
    for k in range(rows_per_tile // CG):
        pltpu.sync_copy(gb0, acc.at[pl.ds(sid * rows_per_tile + k * CG, CG)])
    plsc.subcore_barrier()

    nchb = EBG // CG

    def make_idx(i, ixg, ixw):
        off = i * CG
        for j in range(CG // L):
            o = off + j * L
            ixg[pl.ds(j * L, L)] = srcb[pl.ds(o, L)]
            ixw[0, pl.ds(j * L, L)] = dstb[pl.ds(o, L)]

    def wait_g(gb, ixg, sem):
        pltpu.make_async_copy(tgcn_ref.at[ixg], gb, sem).wait()

    def wait_s(gb, ixw, sem):
        pltpu.make_async_copy(gb, acc.at[ixw.at[0]], sem).wait()

    def block(b, _):
        pltpu.sync_copy(src_ref.at[pl.ds(base + b * EBG, EBG)], srcb)
        pltpu.sync_copy(dst_ref.at[pl.ds(base + b * EBG, EBG)], dstb)

        make_idx(0, ixg0, ixw0)
        pltpu.async_copy(tgcn_ref.at[ixg0], gb0, sem_g0)

        def pair(t, _):
            a = 2 * t

            @pl.when(t > 0)
            def _():
                wait_s(gb1, ixw1, sem_s1)
            make_idx(a + 1, ixg1, ixw1)
            pltpu.async_copy(tgcn_ref.at[ixg1], gb1, sem_g1)
            wait_g(gb0, ixg0, sem_g0)
            pltpu.async_copy(gb0, acc.at[ixw0.at[0]], sem_s0, add=True)
            wait_s(gb0, ixw0, sem_s0)

            @pl.when(a + 2 < nchb)
            def _():
                make_idx(a + 2, ixg0, ixw0)
                pltpu.async_copy(tgcn_ref.at[ixg0], gb0, sem_g0)
            wait_g(gb1, ixg1, sem_g1)
            pltpu.async_copy(gb1, acc.at[ixw1.at[0]], sem_s1, add=True)
            return 0

        lax.fori_loop(0, nchb // 2, pair, 0)
        wait_s(gb1, ixw1, sem_s1)
        return 0

    lax.fori_loop(0, ET32 // EBG, block, 0)
    plsc.subcore_barrier()
    pltpu.sync_copy(acc.at[pl.ds(sid * rows_per_tile, rows_per_tile)],
                    agcn_ref.at[c, pl.ds(sid * rows_per_tile, rows_per_tile)])


def _gcn_pass(src, dst, tgcn):
    f = pl.kernel(
        _gcn_body,
        out_type=jax.ShapeDtypeStruct((NC, NPAD, D), jnp.float32),
        mesh=_sc_mesh(),
        scratch_types=[
            pltpu.VMEM((EBG,), jnp.int32),
            pltpu.VMEM((EBG,), jnp.int32),
            pltpu.VMEM((CG, D), jnp.float32),
            pltpu.VMEM((CG, D), jnp.float32),
            pltpu.VMEM((CG,), jnp.int32),
            pltpu.VMEM((CG,), jnp.int32),
            pltpu.VMEM((1, CG), jnp.int32),
            pltpu.VMEM((1, CG), jnp.int32),
            pltpu.VMEM_SHARED((NPAD, D), jnp.float32),
            pltpu.SemaphoreType.DMA,
            pltpu.SemaphoreType.DMA,
            pltpu.SemaphoreType.DMA,
            pltpu.SemaphoreType.DMA,
        ],
        compiler_params=_SC_PARAMS,
    )
    return f(src, dst, tgcn)


def _gat_body(src_ref, dst_ref, tgat_ref, s_ref, d_ref, agat_ref, sden_ref,
              s_v, d_v, srcb, dstb, gb0, gb1, pv0, pv1, ixg0, ixg1,
              ixw0, ixw1, acc, den_sh,
              sem_g0, sem_g1, sem_s0, sem_s1, sem_d0, sem_d1):
    c = lax.axis_index("c")
    sid = lax.axis_index("s")
    base = sid * ET16

    zeros16 = jnp.zeros((L,), jnp.float32)
    dchunk = 2 * NPAD // NS

    def zs_body(i, _):
        s_v[pl.ds(i * L, L)] = zeros16
        return 0

    lax.fori_loop(0, dchunk // L, zs_body, 0)
    pltpu.sync_copy(s_v.at[pl.ds(0, dchunk)],
                    den_sh.at[pl.ds(sid * dchunk, dchunk)])

    def zb_body(r, _):
        for k in range(HALF // L):
            gb0[r, pl.ds(k * L, L)] = zeros16
        return 0

    lax.fori_loop(0, CA, zb_body, 0)
    rows_per_tile = 2 * NPAD // NS
    for k in range(rows_per_tile // CA):
        pltpu.sync_copy(gb0, acc.at[pl.ds(sid * rows_per_tile + k * CA, CA)])

    pltpu.sync_copy(s_ref, s_v)
    pltpu.sync_copy(d_ref, d_v)
    plsc.subcore_barrier()

    gat_off = c * (2 * NPAD)
    nchb = EBA // CA

    def make_idx(i, ixg, ixw, pv):
        off = i * CA
        for j in range(CA // L):
            o = off + j * L
            sv = srcb[pl.ds(o, L)]
            dv = dstb[pl.ds(o, L)]
            sg = plsc.load_gather(s_v, [sv])
            dg = plsc.load_gather(d_v, [dv])
            negb = (sg + dg) < 0.0
            negi = negb.astype(jnp.int32)
            pv[pl.ds(j * L, L)] = jnp.exp(jnp.where(negb, 0.2 * sg, sg))
            ixg[pl.ds(j * L, L)] = sv + negi * NPAD + gat_off
            ixw[0, pl.ds(j * L, L)] = dv + negi * NPAD

    def wait_g(gb, ixg, sem):
        pltpu.make_async_copy(tgat_ref.at[ixg], gb, sem).wait()

    def wait_sd(gb, pv, ixw, sem_s, sem_d):
        pltpu.make_async_copy(gb, acc.at[ixw.at[0]], sem_s).wait()
        pltpu.make_async_copy(pv, den_sh.at[ixw.at[0]], sem_d).wait()

    def issue_sd(gb, pv, ixw, sem_s, sem_d):
        pltpu.async_copy(gb, acc.at[ixw.at[0]], sem_s, add=True)
        pltpu.async_copy(pv, den_sh.at[ixw.at[0]], sem_d, add=True)

    def block(b, _):
        pltpu.sync_copy(src_ref.at[pl.ds(base + b * EBA, EBA)], srcb)
        pltpu.sync_copy(dst_ref.at[pl.ds(base + b * EBA, EBA)], dstb)

        make_idx(0, ixg0, ixw0, pv0)
        pltpu.async_copy(tgat_ref.at[ixg0], gb0, sem_g0)

        def pair(t, _):
            a = 2 * t

            @pl.when(t > 0)
            def _():
                wait_sd(gb1, pv1, ixw1, sem_s1, sem_d1)
            make_idx(a + 1, ixg1, ixw1, pv1)
            pltpu.async_copy(tgat_ref.at[ixg1], gb1, sem_g1)
            wait_g(gb0, ixg0, sem_g0)
            issue_sd(gb0, pv0, ixw0, sem_s0, sem_d0)
            wait_sd(gb0, pv0, ixw0, sem_s0, sem_d0)

            @pl.when(a + 2 < nchb)
            def _():
                make_idx(a + 2, ixg0, ixw0, pv0)
                pltpu.async_copy(tgat_ref.at[ixg0], gb0, sem_g0)
            wait_g(gb1, ixg1, sem_g1)
            issue_sd(gb1, pv1, ixw1, sem_s1, sem_d1)
            return 0

        lax.fori_loop(0, nchb // 2, pair, 0)
        wait_sd(gb1, pv1, ixw1, sem_s1, sem_d1)
        return 0

    lax.fori_loop(0, ET16 // EBA, block, 0)
    plsc.subcore_barrier()
    pltpu.sync_copy(acc.at[pl.ds(sid * rows_per_tile, rows_per_tile)],
                    agat_ref.at[c, pl.ds(sid * rows_per_tile, rows_per_tile)])
    pltpu.sync_copy(den_sh.at[pl.ds(sid * dchunk, dchunk)],
                    sden_ref.at[c, pl.ds(sid * dchunk, dchunk)])


def _gat_pass(src, dst, tgat_f, svec, dvec):
    f = pl.kernel(
        _gat_body,
        out_type=(
            jax.ShapeDtypeStruct((NC, 2 * NPAD, HALF), jnp.float32),
            jax.ShapeDtypeStruct((NC, 2 * NPAD), jnp.float32),
        ),
        mesh=_sc_mesh(),
        scratch_types=[
            pltpu.VMEM((NPAD,), jnp.float32),
            pltpu.VMEM((NPAD,), jnp.float32),
            pltpu.VMEM((EBA,), jnp.int32),
            pltpu.VMEM((EBA,), jnp.int32),
            pltpu.VMEM((CA, HALF), jnp.float32),
            pltpu.VMEM((CA, HALF), jnp.float32),
            pltpu.VMEM((CA,), jnp.float32),
            pltpu.VMEM((CA,), jnp.float32),
            pltpu.VMEM((CA,), jnp.int32),
            pltpu.VMEM((CA,), jnp.int32),
            pltpu.VMEM((1, CA), jnp.int32),
            pltpu.VMEM((1, CA), jnp.int32),
            pltpu.VMEM_SHARED((2 * NPAD, HALF), jnp.float32),
            pltpu.VMEM_SHARED((2 * NPAD,), jnp.float32),
            pltpu.SemaphoreType.DMA,
            pltpu.SemaphoreType.DMA,
            pltpu.SemaphoreType.DMA,
            pltpu.SemaphoreType.DMA,
            pltpu.SemaphoreType.DMA,
            pltpu.SemaphoreType.DMA,
        ],
        compiler_params=_SC_PARAMS,
    )
    return f(src, dst, tgat_f, svec, dvec)


def _s2a_body(agat_ref, sden_ref, dvec_ref, bgat_ref, y2_ref):
    Apos = jnp.concatenate([agat_ref[0, 0], agat_ref[1, 0]], axis=1)
    Aneg = jnp.concatenate([agat_ref[0, 1], agat_ref[1, 1]], axis=1)
    dv = dvec_ref[...]
    Q = jnp.exp(dv)
    Q2 = jnp.exp(0.2 * dv)
    sden = sden_ref[...]
    denom = jnp.maximum(Q * sden[:, 0:1] + Q2 * sden[:, 1:2], 1e-9)
    y2_ref[...] = (Q * Apos + Q2 * Aneg) / denom + bgat_ref[...]


def _stage2a(agat4, sden_t, dvec, b_gat):
    grid = (NPAD // R1,)
    return pl.pallas_call(
        _s2a_body,
        grid=grid,
        in_specs=[
            pl.BlockSpec((NC, 2, R1, HALF), lambda i: (0, 0, i, 0)),
            pl.BlockSpec((R1, 2), lambda i: (i, 0)),
            pl.BlockSpec((R1, 1), lambda i: (i, 0)),
            pl.BlockSpec((1, D), lambda i: (0, 0)),
        ],
        out_specs=pl.BlockSpec((R1, D), lambda i: (i, 0)),
        out_shape=jax.ShapeDtypeStruct((NPAD, D), jnp.float32),
    )(agat4, sden_t, dvec, b_gat.reshape(1, D))


def _s2b_body(agcn_ref, y2_ref, rin_ref, bgcn_ref, wdec_ref, bdec_ref,
              out_ref):
    y1 = rin_ref[...] * (agcn_ref[0] + agcn_ref[1]) + bgcn_ref[...]
    y2 = y2_ref[...]
    yraw = jnp.concatenate([y1, y2], axis=1)
    y = jnp.dot(yraw, wdec_ref[...], preferred_element_type=jnp.float32) \
        + bdec_ref[...] + y1 + y2
    out_ref[...] = y


def _stage2b(agcn, y2, rin, b_gcn, W_dec, b_dec):
    grid = (NPAD // R1,)
    return pl.pallas_call(
        _s2b_body,
        grid=grid,
        in_specs=[
            pl.BlockSpec((NC, R1, D), lambda i: (0, i, 0)),
            pl.BlockSpec((R1, D), lambda i: (i, 0)),
            pl.BlockSpec((R1, 1), lambda i: (i, 0)),
            pl.BlockSpec((1, D), lambda i: (0, 0)),
            pl.BlockSpec((2 * D, D), lambda i: (0, 0)),
            pl.BlockSpec((1, D), lambda i: (0, 0)),
        ],
        out_specs=pl.BlockSpec((R1, D), lambda i: (i, 0)),
        out_shape=jax.ShapeDtypeStruct((N, D), jnp.float32),
    )(agcn, y2, rin, b_gcn.reshape(1, D), W_dec, b_dec.reshape(1, D))


def kernel(x, edge_index, W_gcn, b_gcn, W_gat, a_src, a_dst, b_gat,
           W_dec, b_dec):
    ei = edge_index.astype(jnp.int32)
    pad_idx = N + jnp.arange(EP - E, dtype=jnp.int32) % (NPAD - N)
    src = jnp.concatenate([ei[0], pad_idx])
    dst = jnp.concatenate([ei[1], pad_idx])
    degp = _deg_partials(src, dst)
    degt = (degp[0] + degp[1]).reshape(2, NPAD).transpose(1, 0)
    xp = jnp.pad(x, ((0, NPAD - N), (0, 0)))
    tgcn, tgat, svec, dvec, rin = _stage1(xp, W_gcn, W_gat, a_src, a_dst, degt)
    tgat_f = tgat.reshape(NC * 2 * NPAD, HALF)
    agat, sden = _gat_pass(src, dst, tgat_f,
                           svec.reshape(NPAD), dvec.reshape(NPAD))
    agcn = _gcn_pass(src, dst, tgcn)
    agat4 = agat.reshape(NC, 2, NPAD, HALF)
    sden_t = sden[0].reshape(2, NPAD).transpose(1, 0)
    y2 = _stage2a(agat4, sden_t, dvec, b_gat)
    return _stage2b(agcn, y2, rin, b_gcn, W_dec, b_dec)
